# baseline (speedup 1.0000x reference)
import functools

import jax
import jax.numpy as jnp
from jax import lax
from jax.experimental import pallas as pl
from jax.experimental.pallas import tpu as pltpu
from jax.experimental.pallas import tpu_sc as plsc

NP_CFG = 10240
NP_AST = 50176
N_CFG_REAL = 10000
N_AST_REAL = 50000
CQ = 2560
K_CFG = 4
AQ = 3584
K_AST = 14
ACC_ROWS = AQ + 128
B_E = 128
RING = 3
PREF = 2
BN = 256


def _sc_mesh():
    return plsc.VectorSubcoreMesh(core_axis_name="c", subcore_axis_name="s")


def _f32(shape):
    return jax.ShapeDtypeStruct(shape, jnp.float32)


def _mm_slabs(x, w1, w2, w3):
    n, din = x.shape
    d = w1.shape[1]
    ns = d // 128

    def body(x_ref, w1_ref, w2_ref, w3_ref, *o_refs):
        xb = x_ref[...]
        for wi, w_ref in enumerate((w1_ref, w2_ref, w3_ref)):
            y = jnp.dot(xb, w_ref[...], preferred_element_type=jnp.float32)
            for s in range(ns):
                o_refs[wi * ns + s][...] = y[:, s * 128:(s + 1) * 128]

    outs = pl.pallas_call(
        body,
        grid=(n // BN,),
        in_specs=[pl.BlockSpec((BN, din), lambda i: (i, 0))]
        + [pl.BlockSpec((din, d), lambda i: (0, 0))] * 3,
        out_specs=[pl.BlockSpec((BN, 128), lambda i: (i, 0))] * (3 * ns),
        out_shape=[_f32((n, 128))] * (3 * ns),
    )(x, w1, w2, w3)
    return outs[0:ns], outs[ns:2 * ns], outs[2 * ns:3 * ns]


def _epilogue_cfg(agg_slabs, deg_ccac, deg_tc, v, s_slabs, b):
    n = deg_ccac.shape[0]
    ns = len(agg_slabs)
    d = ns * 128

    def body(*refs):
        agg_r = refs[0:ns]
        d1_ref, dtc_ref, v_ref = refs[ns], refs[ns + 1], refs[ns + 2]
        s_r = refs[ns + 3:2 * ns + 3]
        b_ref = refs[2 * ns + 3]
        o_ref = refs[2 * ns + 4]
        d1 = d1_ref[:, 0:1]
        dtc = dtc_ref[:, 0:1]
        invd = 1.0 / jnp.maximum(d1 + dtc, 1.0)
        for s in range(ns):
            cs = slice(s * 128, (s + 1) * 128)
            acc = agg_r[s][...] + dtc * v_ref[0:1, cs]
            o_ref[:, cs] = jnp.maximum(
                acc * invd + s_r[s][...] + b_ref[0:1, cs], 0.0)

    return pl.pallas_call(
        body,
        grid=(n // BN,),
        in_specs=[pl.BlockSpec((BN, 128), lambda i: (i, 0))] * ns
        + [pl.BlockSpec((BN, 128), lambda i: (i, 0))] * 2
        + [pl.BlockSpec((8, d), lambda i: (0, 0))]
        + [pl.BlockSpec((BN, 128), lambda i: (i, 0))] * ns
        + [pl.BlockSpec((1, d), lambda i: (0, 0))],
        out_specs=pl.BlockSpec((BN, d), lambda i: (i, 0)),
        out_shape=_f32((n, d)),
    )(*agg_slabs, deg_ccac, deg_tc, v, *s_slabs, b)


def _epilogue_ast(agg_slabs, deg, s_slabs, b):
    n = deg.shape[0]
    ns = len(agg_slabs)
    d = ns * 128

    def body(*refs):
        agg_r = refs[0:ns]
        deg_ref = refs[ns]
        s_r = refs[ns + 1:2 * ns + 1]
        b_ref = refs[2 * ns + 1]
        o_ref = refs[2 * ns + 2]
        invd = 1.0 / jnp.maximum(deg_ref[:, 0:1], 1.0)
        for s in range(ns):
            cs = slice(s * 128, (s + 1) * 128)
            o_ref[:, cs] = jnp.maximum(
                agg_r[s][...] * invd + s_r[s][...] + b_ref[0:1, cs], 0.0)

    return pl.pallas_call(
        body,
        grid=(n // BN,),
        in_specs=[pl.BlockSpec((BN, 128), lambda i: (i, 0))] * ns
        + [pl.BlockSpec((BN, 128), lambda i: (i, 0))]
        + [pl.BlockSpec((BN, 128), lambda i: (i, 0))] * ns
        + [pl.BlockSpec((1, d), lambda i: (0, 0))],
        out_specs=pl.BlockSpec((BN, d), lambda i: (i, 0)),
        out_shape=_f32((n, d)),
    )(*agg_slabs, deg, *s_slabs, b)


def _embed_cfg(glabel, content, w, b):
    n = glabel.shape[0]

    def body(g_ref, c_ref, w_ref, b_ref, o_ref):
        o_ref[:, 0:128] = g_ref[...]
        o_ref[:, 128:256] = (
            jnp.dot(c_ref[...], w_ref[...], preferred_element_type=jnp.float32)
            + b_ref[...]
        )

    return pl.pallas_call(
        body,
        grid=(n // BN,),
        in_specs=[
            pl.BlockSpec((BN, 128), lambda i: (i, 0)),
            pl.BlockSpec((BN, 128), lambda i: (i, 0)),
            pl.BlockSpec((128, 128), lambda i: (0, 0)),
            pl.BlockSpec((1, 128), lambda i: (0, 0)),
        ],
        out_specs=pl.BlockSpec((BN, 256), lambda i: (i, 0)),
        out_shape=_f32((n, 256)),
    )(glabel, content, w, b)


def _embed_ast(glabel, garity, content, w, b):
    n = glabel.shape[0]

    def body(gl_ref, ga_ref, c_ref, w_ref, b_ref, o_ref):
        o_ref[:, 0:128] = gl_ref[...] + ga_ref[...]
        o_ref[:, 128:256] = (
            jnp.dot(c_ref[...], w_ref[...], preferred_element_type=jnp.float32)
            + b_ref[...]
        )

    return pl.pallas_call(
        body,
        grid=(n // BN,),
        in_specs=[
            pl.BlockSpec((BN, 128), lambda i: (i, 0)),
            pl.BlockSpec((BN, 128), lambda i: (i, 0)),
            pl.BlockSpec((BN, 128), lambda i: (i, 0)),
            pl.BlockSpec((128, 128), lambda i: (0, 0)),
            pl.BlockSpec((1, 128), lambda i: (0, 0)),
        ],
        out_specs=pl.BlockSpec((BN, 256), lambda i: (i, 0)),
        out_shape=_f32((n, 256)),
    )(glabel, garity, content, w, b)


def _test_chain(temb, selfs, bs, wtcs):
    douts = [w.shape[1] for w in wtcs]

    def body(t_ref, s0, s1, s2, s3, s4, b0, b1, b2, b3, b4,
             w0, w1, w2, w3, w4, v0, v1, v2, v3, v4):
        def mm(x, w_ref, r0=None, r1=None):
            w = w_ref[...] if r0 is None else w_ref[r0:r1, :]
            return jnp.dot(x, w, preferred_element_type=jnp.float32)

        t = jnp.broadcast_to(t_ref[0:1, :], (8, 256))
        v0[...] = mm(t, w0)
        t1 = jnp.maximum(mm(t, s0) + b0[...], 0.0)
        v1[...] = mm(t1, w1)
        t2 = jnp.maximum(mm(t1, s1) + b1[...], 0.0)
        v2[...] = mm(t1, w2, 0, 256) + mm(t2, w2, 256, 512)
        t3 = jnp.maximum(mm(t1, s2, 0, 256) + mm(t2, s2, 256, 512) + b2[...], 0.0)
        v3[...] = mm(t3, w3)
        t4 = jnp.maximum(mm(t3, s3) + b3[...], 0.0)
        v4[...] = mm(t3, w4, 0, 512) + mm(t4, w4, 512, 1024)

    specs = [pl.BlockSpec(a.shape, lambda i, _r=len(a.shape): (0,) * _r)
             for a in [temb] + list(selfs) + list(bs) + list(wtcs)]
    return pl.pallas_call(
        body,
        grid=(1,),
        in_specs=specs,
        out_specs=[pl.BlockSpec((8, d), lambda i: (0, 0)) for d in douts],
        out_shape=[_f32((8, d)) for d in douts],
    )(temb, *selfs, *bs, *wtcs)


def _decode(h, w_pad, b_pad, n_real, c_real):
    n = h.shape[0]

    def body(h_ref, w_ref, b_ref, o_lg, o_sm):
        z = jnp.dot(h_ref[...], w_ref[...], preferred_element_type=jnp.float32) \
            + b_ref[...]
        o_lg[...] = z
        m = jnp.max(z, axis=1, keepdims=True)
        p = jnp.exp(z - m)
        o_sm[...] = p / jnp.sum(p, axis=1, keepdims=True)

    lg, sm = pl.pallas_call(
        body,
        grid=(n // BN,),
        in_specs=[
            pl.BlockSpec((BN, 256), lambda i: (i, 0)),
            pl.BlockSpec((256, 128), lambda i: (0, 0)),
            pl.BlockSpec((1, 128), lambda i: (0, 0)),
        ],
        out_specs=[pl.BlockSpec((BN, 128), lambda i: (i, 0))] * 2,
        out_shape=[_f32((n, 128))] * 2,
    )(h, w_pad, b_pad)
    return lg[:n_real, :c_real], sm[:n_real, :c_real]


def _prep_edges(src, dst, c_rows, k_chunks):
    e = src.shape[0]
    nbt = -(-(-(-e // 16) // B_E)) + k_chunks + 1
    nbt = -(-nbt // 8) * 8
    src = src.astype(jnp.int32)
    dst = dst.astype(jnp.int32)
    order = jnp.argsort(dst)
    ss = src[order]
    ds = dst[order]
    ck = ds // c_rows
    cstart = jnp.searchsorted(
        ds, jnp.arange(k_chunks, dtype=jnp.int32) * c_rows).astype(jnp.int32)
    cnt = jnp.diff(jnp.concatenate([cstart, jnp.array([e], jnp.int32)]))
    cnt_kt = (cnt[:, None] - jnp.arange(16, dtype=jnp.int32)[None, :] + 15) // 16
    nb_kt = -(-cnt_kt // B_E)
    base_kt = jnp.cumsum(nb_kt, axis=0) - nb_kt
    idx = jnp.arange(e, dtype=jnp.int32)
    r = idx - cstart[ck]
    t = r % 16
    q = r // 16
    pos = (t * nbt + base_kt[ck, t] + q // B_E) * B_E + q % B_E
    tot = 16 * nbt * B_E
    src_p = jnp.zeros((tot,), jnp.int32).at[pos].set(ss)
    dl_p = jnp.full((tot,), c_rows, jnp.int32).at[pos].set(ds - ck * c_rows)
    meta = jnp.zeros((16 * k_chunks, 8, 128), jnp.int32)
    meta = meta.at[:, 0, 0].set(base_kt.T.reshape(-1))
    meta = meta.at[:, 0, 1].set(nb_kt.T.reshape(-1))
    return (src_p.reshape(16, nbt, B_E), dl_p.reshape(16, nbt, B_E),
            meta, nbt)


def _dummy_edges():
    return (jnp.zeros((16, 1, B_E), jnp.int32),
            jnp.full((16, 1, B_E), CQ, jnp.int32),
            jnp.zeros((16 * K_CFG, 8, 128), jnp.int32), 1)


def _dummy_edges_ast():
    return (jnp.zeros((16, 1, B_E), jnp.int32),
            jnp.full((16, 1, B_E), AQ, jnp.int32),
            jnp.zeros((16 * K_AST, 8, 128), jnp.int32), 1)


def _sc_embed_gather(cfg_tab, cfg_idx, astl_tab, astl_idx, asta_tab, asta_idx):
    GB = 32
    nb_cfg = NP_CFG // 32 // GB
    nb_ast = NP_AST // 32 // GB
    mesh = _sc_mesh()

    @functools.partial(
        pl.kernel, mesh=mesh,
        out_type=[_f32((NP_CFG, 128)), _f32((NP_AST, 128)), _f32((NP_AST, 128))],
        scratch_types=[
            pltpu.VMEM((nb_cfg, GB), jnp.int32),
            pltpu.VMEM((nb_ast, GB), jnp.int32),
            pltpu.VMEM((nb_ast, GB), jnp.int32),
            pltpu.VMEM((GB, 128), jnp.float32),
            pltpu.SemaphoreType.DMA,
        ],
    )
    def k(ct, ci, lt, li, at_, ai, o_c, o_l, o_a, bc, bl, ba, rows, sem):
        c = lax.axis_index("c")
        s = lax.axis_index("s")
        wid = s * 2 + c
        pltpu.sync_copy(ci.at[wid], bc)
        pltpu.sync_copy(li.at[wid], bl)
        pltpu.sync_copy(ai.at[wid], ba)

        def gather(tab, buf, out, nb, per_tile):
            def bd(j, _):
                cp = pltpu.make_async_copy(tab.at[buf.at[j]], rows, sem)
                cp.start()
                cp.wait()
                pltpu.sync_copy(rows, out.at[pl.ds(wid * per_tile + j * GB, GB)])
                return 0
            lax.fori_loop(0, nb, bd, 0)

        gather(ct, bc, o_c, nb_cfg, NP_CFG // 32)
        gather(lt, bl, o_l, nb_ast, NP_AST // 32)
        gather(at_, ba, o_a, nb_ast, NP_AST // 32)

    return k(cfg_tab, cfg_idx.reshape(32, nb_cfg, GB),
             astl_tab, astl_idx.reshape(32, nb_ast, GB),
             asta_tab, asta_idx.reshape(32, nb_ast, GB))


def _extract_lane(vec, lane):
    lanes = lax.iota(jnp.int32, 16)
    return jnp.max(jnp.where(lanes == lane, vec, jnp.zeros((16,), jnp.int32)))


def _sc_aggregate(tabs_cc, tabs_ac, tabs_aa, tabs_ca,
                  e_cc, e_ac, e_aa, e_ca, zeros_hbm):
    ns = len(tabs_cc)
    (scc, dcc, mcc, nbt_cc) = e_cc
    (sac, dac, mac, nbt_ac) = e_ac
    (saa, daa, maa, nbt_aa) = e_aa
    (sca, dca, mca, nbt_ca) = e_ca
    k_cc, k_ac = mcc.shape[0] // 16, mac.shape[0] // 16
    k_aa, k_ca = maa.shape[0] // 16, mca.shape[0] // 16
    mesh = _sc_mesh()
    n_in = 4 * ns + 13

    @functools.partial(
        pl.kernel, mesh=mesh,
        out_type=[_f32((NP_CFG, 128))] * ns + [_f32((NP_AST, 128))] * ns,
        scratch_types=[
            pltpu.VMEM((max(nbt_cc, nbt_aa), B_E), jnp.int32),
            pltpu.VMEM((max(nbt_cc, nbt_aa), B_E), jnp.int32),
            pltpu.VMEM((max(nbt_ac, nbt_ca), B_E), jnp.int32),
            pltpu.VMEM((max(nbt_ac, nbt_ca), B_E), jnp.int32),
            pltpu.VMEM((8, 128), jnp.int32),
            pltpu.VMEM((RING, B_E, 128), jnp.float32),
            pltpu.VMEM_SHARED((ACC_ROWS, 128), jnp.float32),
            pltpu.SemaphoreType.DMA((RING,)),
            pltpu.SemaphoreType.DMA((RING,)),
        ],
    )
    def k(*refs):
        ins = refs[0:n_in]
        t_cc, t_ac = ins[0:ns], ins[ns:2 * ns]
        t_aa, t_ca = ins[2 * ns:3 * ns], ins[3 * ns:4 * ns]
        (scc_h, dcc_h, mcc_h, sac_h, dac_h, mac_h,
         saa_h, daa_h, maa_h, sca_h, dca_h, mca_h, zeros_h) = ins[4 * ns:]
        outs = refs[n_in:n_in + 2 * ns]
        o_cfg, o_ast = outs[0:ns], outs[ns:2 * ns]
        (bs0, bd0, bs1, bd1, mrow,
         rows, acc, sem_g, sem_s) = refs[n_in + 2 * ns:]
        c = lax.axis_index("c")
        s = lax.axis_index("s")

        def load_idx(s_h, d_h, nbt, sb, db):
            pltpu.sync_copy(s_h.at[s], sb.at[pl.ds(0, nbt)])
            pltpu.sync_copy(d_h.at[s], db.at[pl.ds(0, nbt)])

        def zero_rows(per_tile):
            pltpu.sync_copy(zeros_h.at[pl.ds(0, per_tile)],
                            acc.at[pl.ds(s * per_tile, per_tile)])

        def ring(t2, sbuf, dbuf, m_h, k_et, nbt_et, ck):
            pltpu.sync_copy(m_h.at[s * k_et + ck], mrow)
            mv = mrow[0, 0:16]
            o = jnp.minimum(jnp.maximum(mv[0], 0), nbt_et - 1)
            nb = jnp.minimum(jnp.maximum(mv[1], 0), nbt_et)

            def bd(j, _):
                cp = pltpu.make_async_copy(t2.at[sbuf.at[o + j]], rows.at[0],
                                           sem_g.at[0])
                cp.start()
                cp.wait()
                pltpu.sync_copy(rows.at[0], acc.at[dbuf.at[o + j]], add=True)
                return 0

            lax.fori_loop(0, nb, bd, 0)

        for sl in range(ns):
            load_idx(scc_h, dcc_h, nbt_cc, bs0, bd0)
            load_idx(sac_h, dac_h, nbt_ac, bs1, bd1)

            def cfg_iter(ki, _):
                ck = ki * 2 + c
                zero_rows((CQ + 128) // 16)
                plsc.subcore_barrier()
                ring(t_cc[sl], bs0, bd0, mcc_h, k_cc, nbt_cc, ck)
                ring(t_ac[sl], bs1, bd1, mac_h, k_ac, nbt_ac, ck)
                plsc.subcore_barrier()
                pltpu.sync_copy(
                    acc.at[pl.ds(s * (CQ // 16), CQ // 16)],
                    o_cfg[sl].at[pl.ds(ck * CQ + s * (CQ // 16), CQ // 16)])
                plsc.subcore_barrier()
                return 0

            lax.fori_loop(0, K_CFG // 2, cfg_iter, 0)

            load_idx(saa_h, daa_h, nbt_aa, bs0, bd0)
            load_idx(sca_h, dca_h, nbt_ca, bs1, bd1)

            def ast_iter(ki, _):
                ck = ki * 2 + c
                zero_rows(ACC_ROWS // 16)
                plsc.subcore_barrier()
                ring(t_aa[sl], bs0, bd0, maa_h, k_aa, nbt_aa, ck)
                ring(t_ca[sl], bs1, bd1, mca_h, k_ca, nbt_ca, ck)
                plsc.subcore_barrier()
                pltpu.sync_copy(
                    acc.at[pl.ds(s * (AQ // 16), AQ // 16)],
                    o_ast[sl].at[pl.ds(ck * AQ + s * (AQ // 16), AQ // 16)])
                plsc.subcore_barrier()
                return 0

            lax.fori_loop(0, K_AST // 2, ast_iter, 0)

    outs = k(*tabs_cc, *tabs_ac, *tabs_aa, *tabs_ca,
             scc, dcc, mcc, sac, dac, mac, saa, daa, maa, sca, dca, mca,
             zeros_hbm)
    return outs[0:ns], outs[ns:2 * ns]


def _agg_fake_one(tabs, e, n_dst, c_rows):
    s3, d3, meta, nbt = e
    kk = meta.shape[0] // 16
    t = jnp.concatenate(tabs, axis=1)
    base = meta[:, 0, 0].reshape(16, kk)
    src = s3.reshape(16, -1)
    dl = d3.reshape(16, -1)
    bi = jnp.arange(nbt, dtype=jnp.int32)
    kmap = jnp.sum((bi[None, :, None] >= base[:, None, :]).astype(jnp.int32),
                   axis=2) - 1
    kmap = jnp.clip(kmap, 0, kk - 1)
    kfull = jnp.repeat(kmap, B_E, axis=1)
    gdst = kfull * c_rows + dl
    gdst = jnp.where(dl < c_rows, gdst, n_dst)
    msg = t[src.reshape(-1)]
    out = jax.ops.segment_sum(msg, gdst.reshape(-1), num_segments=n_dst + 1)
    return out[:n_dst]


def _sc_aggregate_fake(tabs_cc, tabs_ac, tabs_aa, tabs_ca,
                       e_cc, e_ac, e_aa, e_ca, zeros_hbm):
    ns = len(tabs_cc)
    agg_cfg = (_agg_fake_one(tabs_cc, e_cc, NP_CFG, CQ)
               + _agg_fake_one(tabs_ac, e_ac, NP_CFG, CQ))
    agg_ast = (_agg_fake_one(tabs_aa, e_aa, NP_AST, AQ)
               + _agg_fake_one(tabs_ca, e_ca, NP_AST, AQ))
    return ([agg_cfg[:, s * 128:(s + 1) * 128] for s in range(ns)],
            [agg_ast[:, s * 128:(s + 1) * 128] for s in range(ns)])


def kernel(cfg_label, cfg_content, ast_label, ast_arity, ast_content,
           cc_src, cc_dst, aa_src, aa_dst, ca_src, ca_dst,
           ac_src, ac_dst, tc_src, tc_dst, params):
    p = params

    zeros_hbm = jnp.zeros((ACC_ROWS // 16, 128), jnp.float32)

    cfg_lab_p = jnp.concatenate(
        [cfg_label.astype(jnp.int32), jnp.zeros((NP_CFG - N_CFG_REAL,), jnp.int32)])
    ast_lab_p = jnp.concatenate(
        [ast_label.astype(jnp.int32), jnp.zeros((NP_AST - N_AST_REAL,), jnp.int32)])
    ast_ari_p = jnp.concatenate(
        [ast_arity.astype(jnp.int32), jnp.zeros((NP_AST - N_AST_REAL,), jnp.int32)])
    cfg_cont_p = jnp.pad(cfg_content, ((0, NP_CFG - N_CFG_REAL), (0, 0)))
    ast_cont_p = jnp.pad(ast_content, ((0, NP_AST - N_AST_REAL), (0, 0)))

    e_cc = _prep_edges(cc_src, cc_dst, CQ, K_CFG)
    e_ac = _prep_edges(ac_src, ac_dst, CQ, K_CFG)
    e_tc = _prep_edges(tc_src, tc_dst, CQ, K_CFG)
    e_aa = _prep_edges(aa_src, aa_dst, AQ, K_AST)
    e_ca = _prep_edges(ca_src, ca_dst, AQ, K_AST)
    e_nil = _dummy_edges()

    g_cfg, g_astl, g_asta = _sc_embed_gather(
        p["cfg_label_emb"], cfg_lab_p, p["ast_label_emb"], ast_lab_p,
        p["ast_arity_emb"], ast_ari_p)
    ones_cfg = jnp.ones((NP_CFG, 128), jnp.float32)
    ones_ast = jnp.ones((NP_AST, 128), jnp.float32)
    (deg_ccac,), (deg_ast,) = _sc_aggregate(
        [ones_cfg], [ones_ast], [ones_ast], [ones_cfg],
        e_cc, e_ac, e_aa, e_ca, zeros_hbm)
    (deg_tc,), _ = _sc_aggregate(
        [ones_cfg], [ones_cfg], [ones_ast], [ones_cfg],
        e_tc, e_nil, _dummy_edges_ast(), e_nil, zeros_hbm)

    h_cfg = _embed_cfg(g_cfg, cfg_cont_p, p["cfg_content_W"],
                       p["cfg_content_b"].reshape(1, 128))
    h_ast = _embed_ast(g_astl, g_asta, ast_cont_p, p["ast_content_W"],
                       p["ast_content_b"].reshape(1, 128))

    vs = _test_chain(
        p["test_emb"].reshape(1, 256),
        [p["l%d_self_test" % i] for i in range(5)],
        [p["l%d_b_test" % i].reshape(1, -1) for i in range(5)],
        [p["l%d_W_tc" % i] for i in range(5)],
    )

    def layer(li, h_cfg, h_ast):
        d = p["l%d_W_cc" % li].shape[1]
        t_cc, t_ca, s_cfg = _mm_slabs(h_cfg, p["l%d_W_cc" % li],
                                      p["l%d_W_ca" % li], p["l%d_self_cfg" % li])
        t_aa, t_ac, s_ast = _mm_slabs(h_ast, p["l%d_W_aa" % li],
                                      p["l%d_W_ac" % li], p["l%d_self_ast" % li])
        agg_cfg, agg_ast = _sc_aggregate(
            t_cc, t_ac, t_aa, t_ca, e_cc, e_ac, e_aa, e_ca, zeros_hbm)
        h_cfg_n = _epilogue_cfg(agg_cfg, deg_ccac, deg_tc, vs[li], s_cfg,
                                p["l%d_b_cfg" % li].reshape(1, d))
        h_ast_n = _epilogue_ast(agg_ast, deg_ast, s_ast,
                                p["l%d_b_ast" % li].reshape(1, d))
        return h_cfg_n, h_ast_n

    h_cfg, h_ast = layer(0, h_cfg, h_ast)
    sk_cfg, sk_ast = h_cfg, h_ast
    h_cfg, h_ast = layer(1, h_cfg, h_ast)
    h_cfg = jnp.concatenate([sk_cfg, h_cfg], axis=1)
    h_ast = jnp.concatenate([sk_ast, h_ast], axis=1)
    h_cfg, h_ast = layer(2, h_cfg, h_ast)
    sk_cfg, sk_ast = h_cfg, h_ast
    h_cfg, h_ast = layer(3, h_cfg, h_ast)
    h_cfg = jnp.concatenate([sk_cfg, h_cfg], axis=1)
    h_ast = jnp.concatenate([sk_ast, h_ast], axis=1)
    h_cfg, h_ast = layer(4, h_cfg, h_ast)

    ncls = p["dec_W"].shape[1]
    ncls_a = p["ast_dec_W"].shape[1]
    dec_w = jnp.pad(p["dec_W"], ((0, 0), (0, 128 - ncls)))
    dec_b = jnp.concatenate(
        [p["dec_b"], jnp.full((128 - ncls,), -1e9, jnp.float32)]).reshape(1, 128)
    adec_w = jnp.pad(p["ast_dec_W"], ((0, 0), (0, 128 - ncls_a)))
    adec_b = jnp.concatenate(
        [p["ast_dec_b"], jnp.full((128 - ncls_a,), -1e9, jnp.float32)]).reshape(1, 128)
    cfg_lg, cfg_sm = _decode(h_cfg, dec_w, dec_b, N_CFG_REAL, ncls)
    ast_lg, ast_sm = _decode(h_ast, adec_w, adec_b, N_AST_REAL, ncls_a)
    return (cfg_lg, cfg_sm, ast_lg, ast_sm)
